# baseline (speedup 1.0000x reference)
import jax
import jax.numpy as jnp
from jax import lax
from jax.experimental import pallas as pl
from jax.experimental.pallas import tpu as pltpu
from jax.experimental.pallas import tpu_sc as plsc

D_MODEL = 1024
N_FEATURES = 131072
BATCH = 4096
K = 64
TOTAL_IDX = BATCH * K
DEATH_THRESHOLD = 0.01

NS = 16
LANES = 16
PER_TILE_IDX = TOTAL_IDX // NS
BINS2 = N_FEATURES // NS
ZEROS_N = N_FEATURES // NS

CH = 4
CHN = PER_TILE_IDX // CH

_NBLK = 8
_BLK = BATCH // _NBLK


def _tc_body(params, x_ref, fm_ref, fsm_ref, means_out, sq_out, norm_out):
    i = pl.program_id(0)
    blk = x_ref[...]
    sq = blk * blk
    csum = jnp.sum(blk, axis=0, keepdims=True)
    cssum = jnp.sum(sq, axis=0, keepdims=True)
    nsum = jnp.sum(jnp.sqrt(jnp.sum(sq, axis=1, keepdims=True)))

    @pl.when(i == 0)
    def _():
        means_out[...] = csum
        sq_out[...] = cssum
        norm_out[0, 0] = nsum

    @pl.when(i > 0)
    def _():
        means_out[...] += csum
        sq_out[...] += cssum
        norm_out[0, 0] += nsum

    @pl.when(i == _NBLK - 1)
    def _():
        wf = params[0]
        scale = params[1]
        means_out[...] = fm_ref[...] * wf + means_out[...] * scale
        sq_out[...] = fsm_ref[...] * wf + sq_out[...] * scale
        norm_out[0, 0] = params[2] * wf + norm_out[0, 0] * scale


def _tc_stats(params_tc, x, feature_means, feature_square_means):
    return pl.pallas_call(
        _tc_body,
        grid=(_NBLK,),
        in_specs=[
            pl.BlockSpec(memory_space=pltpu.SMEM),
            pl.BlockSpec((_BLK, D_MODEL), lambda i: (i, 0)),
            pl.BlockSpec((1, D_MODEL), lambda i: (0, 0)),
            pl.BlockSpec((1, D_MODEL), lambda i: (0, 0)),
        ],
        out_specs=[
            pl.BlockSpec((1, D_MODEL), lambda i: (0, 0)),
            pl.BlockSpec((1, D_MODEL), lambda i: (0, 0)),
            pl.BlockSpec(memory_space=pltpu.SMEM),
        ],
        out_shape=[
            jax.ShapeDtypeStruct((1, D_MODEL), jnp.float32),
            jax.ShapeDtypeStruct((1, D_MODEL), jnp.float32),
            jax.ShapeDtypeStruct((1, 1), jnp.float32),
        ],
        compiler_params=pltpu.CompilerParams(
            dimension_semantics=("arbitrary",)),
    )(params_tc, x, feature_means.reshape(1, D_MODEL),
      feature_square_means.reshape(1, D_MODEL))


def _sc_body(idx_hbm, w_hbm, zeros_hbm,
             cnt_out, nd_out,
             hist,
             idx_v0, idx_v1, idx_v2, idx_v3,
             w_v0, w_v1, w_v2, w_v3,
             nd_v0, nd_v1, nd_v2, nd_v3,
             ones_v,
             semi, semw, semz, sem0):
    idx_vs = [idx_v0, idx_v1, idx_v2, idx_v3]
    w_vs = [w_v0, w_v1, w_v2, w_v3]
    nd_vs = [nd_v0, nd_v1, nd_v2, nd_v3]
    c = lax.axis_index("c")
    s = lax.axis_index("s")

    ibase = s * PER_TILE_IDX
    d_idx = [pltpu.async_copy(idx_hbm.at[pl.ds(ibase + k * CHN, CHN)],
                              idx_vs[k], semi) for k in range(CH)]
    zbase = s * ZEROS_N
    d_z = pltpu.async_copy(zeros_hbm, hist.at[pl.ds(zbase, ZEROS_N)], semz)
    gbase = s * BINS2

    one16 = jnp.full((LANES,), 1, dtype=jnp.int32)
    z16 = jnp.zeros((LANES,), dtype=jnp.int32)
    thr16 = jnp.full((LANES,), DEATH_THRESHOLD, dtype=jnp.float32)

    @pl.when(c == 0)
    def _():
        @pl.loop(0, CHN // LANES, unroll=8)
        def _(j):
            ones_v[pl.ds(j * LANES, LANES)] = one16

        d_z.wait()
        plsc.subcore_barrier()
        for d in d_idx:
            d.wait()
        descs = [pltpu.async_copy(ones_v, hist.at[idx_vs[k]], sem0,
                                  add=True) for k in range(CH)]
        for d in descs:
            d.wait()
        plsc.subcore_barrier()
        pltpu.sync_copy(hist.at[pl.ds(gbase, BINS2)],
                        cnt_out.at[pl.ds(gbase, BINS2)])

    @pl.when(c == 1)
    def _():
        d_w = [pltpu.async_copy(w_hbm.at[pl.ds(ibase + k * CHN, CHN)],
                                w_vs[k], semw) for k in range(CH)]
        descs = []
        for k in range(CH):
            d_w[k].wait()
            wk, ndk = w_vs[k], nd_vs[k]

            @pl.loop(0, CHN // LANES, unroll=8)
            def _(j, wk=wk, ndk=ndk):
                sl = pl.ds(j * LANES, LANES)
                w16 = wk[sl]
                ndk[sl] = jnp.where(w16 > thr16, one16, z16)

            if k == 0:
                d_z.wait()
                plsc.subcore_barrier()
                for d in d_idx:
                    d.wait()
            descs.append(pltpu.async_copy(nd_vs[k], hist.at[idx_vs[k]],
                                          sem0, add=True))
        for d in descs:
            d.wait()
        plsc.subcore_barrier()
        pltpu.sync_copy(hist.at[pl.ds(gbase, BINS2)],
                        nd_out.at[pl.ds(gbase, BINS2)])


_sc_hist = pl.kernel(
    _sc_body,
    out_type=[
        jax.ShapeDtypeStruct((N_FEATURES,), jnp.int32),
        jax.ShapeDtypeStruct((N_FEATURES,), jnp.int32),
    ],
    mesh=plsc.VectorSubcoreMesh(core_axis_name="c", subcore_axis_name="s"),
    scratch_types=[
        pltpu.VMEM_SHARED((N_FEATURES,), jnp.int32),
        pltpu.VMEM((CHN,), jnp.int32),
        pltpu.VMEM((CHN,), jnp.int32),
        pltpu.VMEM((CHN,), jnp.int32),
        pltpu.VMEM((CHN,), jnp.int32),
        pltpu.VMEM((CHN,), jnp.float32),
        pltpu.VMEM((CHN,), jnp.float32),
        pltpu.VMEM((CHN,), jnp.float32),
        pltpu.VMEM((CHN,), jnp.float32),
        pltpu.VMEM((CHN,), jnp.int32),
        pltpu.VMEM((CHN,), jnp.int32),
        pltpu.VMEM((CHN,), jnp.int32),
        pltpu.VMEM((CHN,), jnp.int32),
        pltpu.VMEM((CHN,), jnp.int32),
        pltpu.SemaphoreType.DMA,
        pltpu.SemaphoreType.DMA,
        pltpu.SemaphoreType.DMA,
        pltpu.SemaphoreType.DMA,
    ],
)


def _upd_body(params, fd_ref, ai_ref, cnt_ref, nd_ref, fd_out, ai_out):
    wf = params[0]
    scale = params[1]
    cnt = cnt_ref[...].astype(jnp.float32)
    fd_out[...] = fd_ref[...] * wf + cnt * scale
    nd = nd_ref[...]
    ai_out[...] = jnp.where(nd > 0, jnp.uint32(0), ai_ref[...] + jnp.uint32(1))


def _tc_update(params_tc, feature_density, activated_in, cnt_h, nd_h):
    return pl.pallas_call(
        _upd_body,
        in_specs=[
            pl.BlockSpec(memory_space=pltpu.SMEM),
            pl.BlockSpec((N_FEATURES,), lambda: (0,)),
            pl.BlockSpec((N_FEATURES,), lambda: (0,)),
            pl.BlockSpec((N_FEATURES,), lambda: (0,)),
            pl.BlockSpec((N_FEATURES,), lambda: (0,)),
        ],
        out_specs=[
            pl.BlockSpec((N_FEATURES,), lambda: (0,)),
            pl.BlockSpec((N_FEATURES,), lambda: (0,)),
        ],
        out_shape=[
            jax.ShapeDtypeStruct((N_FEATURES,), jnp.float32),
            jax.ShapeDtypeStruct((N_FEATURES,), jnp.uint32),
        ],
    )(params_tc, feature_density, activated_in, cnt_h, nd_h)


def kernel(x, k_weights, k_indices, feature_density, activated_in,
           feature_means, feature_square_means, avg_norm, n_steps):
    wf = (n_steps / (n_steps + 1)).astype(jnp.float32)
    nwf = (1 / (n_steps + 1)).astype(jnp.float32)

    params_tc = jnp.stack([wf, nwf / BATCH, avg_norm])
    means2, sq2, norm11 = _tc_stats(params_tc, x, feature_means,
                                    feature_square_means)

    zeros_host = jnp.zeros((ZEROS_N,), dtype=jnp.int32)
    cnt_h, nd_h = _sc_hist(k_indices.reshape(TOTAL_IDX),
                           k_weights.reshape(TOTAL_IDX), zeros_host)
    fd_new, ai_new = _tc_update(params_tc, feature_density, activated_in,
                                cnt_h, nd_h)

    return (norm11[0, 0], means2.reshape(D_MODEL), sq2.reshape(D_MODEL),
            fd_new, ai_new)
